# baseline (speedup 1.0000x reference)
import functools

import jax
import jax.numpy as jnp
from jax import lax
from jax.experimental import pallas as pl
from jax.experimental.pallas import tpu as pltpu
from jax.experimental.pallas import tpu_sc as plsc

C, G, D = 32, 2048, 256
GENE_V = 60000
_NC, _NS = 2, 16
_NW = _NC * _NS
_CH = 128


def _silu(x):
    return x * jax.nn.sigmoid(x)


def _sc_gather(gene_table_i32, gene_idx, cond_table, cond_idx):
    B = gene_idx.shape[0]
    CB = cond_idx.shape[0]
    H = D // 2
    rows_w = B // _NW
    nch = rows_w // _CH

    mesh = plsc.VectorSubcoreMesh(core_axis_name="c", subcore_axis_name="s")

    @functools.partial(
        pl.kernel,
        out_type=(
            jax.ShapeDtypeStruct((B, H), jnp.int32),
            jax.ShapeDtypeStruct((CB, D), jnp.int32),
        ),
        mesh=mesh,
        scratch_types=[
            pltpu.VMEM((rows_w,), jnp.int32),
            pltpu.VMEM((_CH, D), jnp.int32),
            pltpu.VMEM((_CH, D), jnp.int32),
            pltpu.VMEM((_CH, H), jnp.int32),
            pltpu.VMEM((_CH, H), jnp.int32),
            pltpu.VMEM((CB,), jnp.int32),
            pltpu.SemaphoreType.DMA,
            pltpu.SemaphoreType.DMA,
            pltpu.SemaphoreType.DMA,
            pltpu.SemaphoreType.DMA,
            pltpu.SemaphoreType.DMA,
        ],
    )
    def gather_k(table_h, idx_h, ctab_h, cidx_h, out_h, cout_h,
                 idx_v, rows_a, rows_b, pk_a, pk_b, cidx_v,
                 gsem_a, gsem_b, ssem_a, ssem_b, csem):
        wid = lax.axis_index("s") * _NC + lax.axis_index("c")
        base = wid * rows_w
        pltpu.sync_copy(idx_h.at[pl.ds(base, rows_w)], idx_v)

        bufs = (rows_a, rows_b)
        pks = (pk_a, pk_b)
        gsems = (gsem_a, gsem_b)
        ssems = (ssem_a, ssem_b)
        gcp = [None, None]
        scp = [None, None]

        def pack_chunk(rows_v, pk_v):
            rnd = jnp.int32(0x8000)
            msk = jnp.int32(-65536)
            s16 = jnp.int32(16)

            def rowfn(r, carry):
                for i in range(8):
                    ul = rows_v[r, pl.ds(16 * i, 16)]
                    uh = rows_v[r, pl.ds(D // 2 + 16 * i, 16)]
                    w = (lax.shift_right_logical(ul + rnd, s16)
                         | ((uh + rnd) & msk))
                    pk_v[r, pl.ds(16 * i, 16)] = w
                return carry

            lax.fori_loop(0, _CH, rowfn, 0)

        for k in range(2):
            gcp[k] = pltpu.async_copy(
                table_h.at[idx_v.at[pl.ds(k * _CH, _CH)]], bufs[k], gsems[k])
        for k in range(nch):
            b = k % 2
            gcp[b].wait()
            if scp[b] is not None:
                scp[b].wait()
            pack_chunk(bufs[b], pks[b])
            scp[b] = pltpu.async_copy(
                pks[b], out_h.at[pl.ds(base + k * _CH, _CH)], ssems[b])
            if k + 2 < nch:
                gcp[b] = pltpu.async_copy(
                    table_h.at[idx_v.at[pl.ds((k + 2) * _CH, _CH)]],
                    bufs[b], gsems[b])
        scp[0].wait()
        scp[1].wait()

        @pl.when(wid == 0)
        def _():
            pltpu.sync_copy(cidx_h, cidx_v)
            pltpu.async_copy(ctab_h.at[cidx_v], rows_a, csem).wait()
            pltpu.sync_copy(rows_a, cout_h)

    return gather_k(gene_table_i32, gene_idx, cond_table, cond_idx)


def _cell_body(gene_ref, ev_ref, tf_ref, validt_ref, ce_ref,
               W1cat_ref, b1cat_ref, eW2_ref, eb2_ref, eW3_ref, eb3_ref,
               mW2_ref, mb2_ref,
               tf_tab_ref, zero_ref,
               ctxW1_ref, ctxb1_ref, ctxW2_ref, ctxb2_ref,
               pW1_ref, pb1_ref, pW2_ref, pb2_ref,
               xW1_ref, xb1_ref, xW2_ref, xb2_ref,
               prefix_type_ref, rms_ref, pos_ref, out_ref):
    f32 = jnp.float32
    bf16 = jnp.bfloat16
    gu = lax.bitcast_convert_type(gene_ref[0], jnp.uint32)
    gleft = lax.bitcast_convert_type(
        lax.shift_left(gu, jnp.uint32(16)), f32)
    gright = lax.bitcast_convert_type(gu & jnp.uint32(0xFFFF0000), f32)
    gene = jnp.concatenate([gleft, gright], axis=1)
    v = ev_ref[0]
    s = _silu(v * W1cat_ref[...] + b1cat_ref[...])
    h = s[:, :D]
    m = s[:, D:]
    h = _silu(jnp.dot(h.astype(bf16), eW2_ref[...],
                      preferred_element_type=f32) + eb2_ref[...])
    expr = jnp.dot(h.astype(bf16), eW3_ref[...],
                   preferred_element_type=f32) + eb3_ref[...]
    expr = jnp.where(v == 0.0, zero_ref[...], expr)
    mod = jnp.dot(m.astype(bf16), mW2_ref[...],
                  preferred_element_type=f32) + mb2_ref[...]
    scale = jax.nn.sigmoid(mod[:, :D])
    shift = mod[:, D:]
    t0 = tf_tab_ref[0:1, :]
    t1 = tf_tab_ref[1:2, :]
    tf_emb = t0 + tf_ref[0] * (t1 - t0)
    tokens = (gene + expr + tf_emb) * scale + shift
    vrow = validt_ref[0]
    pooled = jnp.dot(vrow, tokens, preferred_element_type=f32) / jnp.maximum(
        jnp.sum(vrow), 1.0)
    ctx = jnp.dot(_silu(jnp.dot(pooled, ctxW1_ref[...],
                                preferred_element_type=f32) + ctxb1_ref[...]),
                  ctxW2_ref[...], preferred_element_type=f32) + ctxb2_ref[...]
    ce = ce_ref[0]
    ptok = jnp.dot(_silu(jnp.dot(ce, pW1_ref[...],
                                 preferred_element_type=f32) + pb1_ref[...]),
                   pW2_ref[...], preferred_element_type=f32) + pb2_ref[...]
    cbias = jnp.dot(_silu(jnp.dot(ce, xW1_ref[...],
                                  preferred_element_type=f32) + xb1_ref[...]),
                    xW2_ref[...], preferred_element_type=f32) + xb2_ref[...]
    prefix_row = ptok + ctx + prefix_type_ref[...] + pos_ref[0:1, :]
    genes = tokens + cbias + pos_ref[1:, :]
    full = jnp.concatenate([prefix_row, genes], axis=0)
    norm = full * lax.rsqrt(
        jnp.mean(full * full, axis=-1, keepdims=True) + 1e-6) * rms_ref[...]
    out_ref[0] = norm


def kernel(expression_values, gene_table, zero_embedding, eW1, eb1, eW2, eb2,
           eW3, eb3, mW1, mb1, mW2, mb2, cond_table, pW1, pb1, pW2, pb2,
           xW1, xb1, xW2, xb2, ctxW1, ctxb1, ctxW2, ctxb2, tf_table,
           pos_table, prefix_type, gene_type, rms_w, input_ids,
           condition_ids, padding_mask, non_tf_mask):
    gene_idx = input_ids.reshape(-1).astype(jnp.int32)
    cond_idx = condition_ids.reshape(-1).astype(jnp.int32)
    gene_table_i32 = lax.bitcast_convert_type(gene_table, jnp.int32)
    cond_table_i32 = lax.bitcast_convert_type(cond_table, jnp.int32)
    gathered, ce_rows_i32 = _sc_gather(gene_table_i32, gene_idx,
                                       cond_table_i32, cond_idx)
    gathered = gathered.reshape(C, G, D // 2)
    ce_rows = lax.bitcast_convert_type(ce_rows_i32, jnp.float32)
    ce3 = ce_rows.reshape(C, 1, 4 * D)

    ev3 = expression_values.reshape(C, G, 1)
    tf3 = non_tf_mask.astype(jnp.float32).reshape(C, G, 1)
    validt = (~padding_mask).astype(jnp.float32).reshape(C, 1, G)
    pos = pos_table[: G + 1]

    row = lambda b: b.reshape(1, -1)
    bf = lambda w: w.astype(jnp.bfloat16)
    W1cat = jnp.concatenate([eW1, mW1], axis=1)
    b1cat = jnp.concatenate([eb1, mb1]).reshape(1, -1)
    tf_tab_adj = tf_table + gene_type.reshape(1, D)
    weights = (W1cat, b1cat, bf(eW2), row(eb2), bf(eW3), row(eb3),
               bf(mW2), row(mb2),
               tf_tab_adj, row(zero_embedding),
               ctxW1, row(ctxb1), ctxW2, row(ctxb2),
               pW1, row(pb1), pW2, row(pb2),
               xW1, row(xb1), xW2, row(xb2),
               prefix_type.reshape(1, D), row(rms_w), pos)

    full_spec = lambda a: pl.BlockSpec(a.shape, lambda c: (0,) * a.ndim)
    in_specs = [
        pl.BlockSpec((1, G, D // 2), lambda c: (c, 0, 0)),
        pl.BlockSpec((1, G, 1), lambda c: (c, 0, 0)),
        pl.BlockSpec((1, G, 1), lambda c: (c, 0, 0)),
        pl.BlockSpec((1, 1, G), lambda c: (c, 0, 0)),
        pl.BlockSpec((1, 1, 4 * D), lambda c: (c, 0, 0)),
    ] + [full_spec(w) for w in weights]

    out = pl.pallas_call(
        _cell_body,
        grid=(C,),
        in_specs=in_specs,
        out_specs=pl.BlockSpec((1, G + 1, D), lambda c: (c, 0, 0)),
        out_shape=jax.ShapeDtypeStruct((C, G + 1, D), jnp.float32),
    )(gathered, ev3, tf3, validt, ce3, *weights)
    return out


# speedup vs baseline: 1.2466x; 1.2466x over previous
import functools

import jax
import jax.numpy as jnp
from jax import lax
from jax.experimental import pallas as pl
from jax.experimental.pallas import tpu as pltpu
from jax.experimental.pallas import tpu_sc as plsc

C, G, D = 32, 2048, 256
GENE_V = 60000
_NC, _NS = 2, 16
_NW = _NC * _NS
_CH = 128


def _silu(x):
    return x * jax.nn.sigmoid(x)


def _sc_gather(gene_table, gene_idx, cond_table, cond_idx):
    B = gene_idx.shape[0]
    CB = cond_idx.shape[0]
    rows_w = B // _NW
    nch = rows_w // _CH

    mesh = plsc.VectorSubcoreMesh(core_axis_name="c", subcore_axis_name="s")

    @functools.partial(
        pl.kernel,
        out_type=(
            jax.ShapeDtypeStruct((B, D), jnp.float32),
            jax.ShapeDtypeStruct((CB, D), jnp.float32),
        ),
        mesh=mesh,
        scratch_types=[
            pltpu.VMEM((rows_w,), jnp.int32),
            pltpu.VMEM((_CH, D), jnp.float32),
            pltpu.VMEM((_CH, D), jnp.float32),
            pltpu.VMEM((CB,), jnp.int32),
            pltpu.VMEM((CB, D), jnp.float32),
            pltpu.SemaphoreType.DMA,
            pltpu.SemaphoreType.DMA,
            pltpu.SemaphoreType.DMA,
            pltpu.SemaphoreType.DMA,
            pltpu.SemaphoreType.DMA,
        ],
    )
    def gather_k(table_h, idx_h, ctab_h, cidx_h, out_h, cout_h,
                 idx_v, rows_a, rows_b, cidx_v, crows_v,
                 gsem_a, gsem_b, ssem_a, ssem_b, csem):
        wid = lax.axis_index("s") * _NC + lax.axis_index("c")
        base = wid * rows_w
        pltpu.sync_copy(idx_h.at[pl.ds(base, rows_w)], idx_v)

        bufs = (rows_a, rows_b)
        gsems = (gsem_a, gsem_b)
        ssems = (ssem_a, ssem_b)
        gcp = [None, None]
        scp = [None, None]
        for k in range(2):
            gcp[k] = pltpu.async_copy(
                table_h.at[idx_v.at[pl.ds(k * _CH, _CH)]], bufs[k], gsems[k])
        for k in range(nch):
            b = k % 2
            gcp[b].wait()
            scp[b] = pltpu.async_copy(
                bufs[b], out_h.at[pl.ds(base + k * _CH, _CH)], ssems[b])
            if k + 2 < nch:
                scp[b].wait()
                gcp[b] = pltpu.async_copy(
                    table_h.at[idx_v.at[pl.ds((k + 2) * _CH, _CH)]],
                    bufs[b], gsems[b])
        scp[0].wait()
        scp[1].wait()

        @pl.when(wid == 0)
        def _():
            pltpu.sync_copy(cidx_h, cidx_v)
            pltpu.async_copy(ctab_h.at[cidx_v], crows_v, csem).wait()
            pltpu.sync_copy(crows_v, cout_h)

    return gather_k(gene_table, gene_idx, cond_table, cond_idx)


def _cell_body(gene_ref, ev_ref, tf_ref, validt_ref, ce_ref,
               W1cat_ref, b1cat_ref, eW2_ref, eb2_ref, eW3_ref, eb3_ref,
               mW2_ref, mb2_ref,
               tf_tab_ref, zero_ref,
               ctxW1_ref, ctxb1_ref, ctxW2_ref, ctxb2_ref,
               pW1_ref, pb1_ref, pW2_ref, pb2_ref,
               xW1_ref, xb1_ref, xW2_ref, xb2_ref,
               prefix_type_ref, rms_ref, pos_ref, out_ref):
    f32 = jnp.float32
    bf16 = jnp.bfloat16
    v = ev_ref[0]
    s = _silu(v * W1cat_ref[...] + b1cat_ref[...])
    h = s[:, :D]
    m = s[:, D:]
    h = _silu(jnp.dot(h.astype(bf16), eW2_ref[...],
                      preferred_element_type=f32) + eb2_ref[...])
    expr = jnp.dot(h.astype(bf16), eW3_ref[...],
                   preferred_element_type=f32) + eb3_ref[...]
    expr = jnp.where(v == 0.0, zero_ref[...], expr)
    mod = jnp.dot(m.astype(bf16), mW2_ref[...],
                  preferred_element_type=f32) + mb2_ref[...]
    scale = jax.nn.sigmoid(mod[:, :D])
    shift = mod[:, D:]
    t0 = tf_tab_ref[0:1, :]
    t1 = tf_tab_ref[1:2, :]
    tf_emb = t0 + tf_ref[0] * (t1 - t0)
    tokens = (gene_ref[0] + expr + tf_emb) * scale + shift
    vrow = validt_ref[0]
    pooled = jnp.dot(vrow, tokens, preferred_element_type=f32) / jnp.maximum(
        jnp.sum(vrow), 1.0)
    ctx = jnp.dot(_silu(jnp.dot(pooled, ctxW1_ref[...],
                                preferred_element_type=f32) + ctxb1_ref[...]),
                  ctxW2_ref[...], preferred_element_type=f32) + ctxb2_ref[...]
    ce = ce_ref[0]
    ptok = jnp.dot(_silu(jnp.dot(ce, pW1_ref[...],
                                 preferred_element_type=f32) + pb1_ref[...]),
                   pW2_ref[...], preferred_element_type=f32) + pb2_ref[...]
    cbias = jnp.dot(_silu(jnp.dot(ce, xW1_ref[...],
                                  preferred_element_type=f32) + xb1_ref[...]),
                    xW2_ref[...], preferred_element_type=f32) + xb2_ref[...]
    prefix_row = ptok + ctx + prefix_type_ref[...] + pos_ref[0:1, :]
    genes = tokens + cbias + pos_ref[1:, :]
    full = jnp.concatenate([prefix_row, genes], axis=0)
    norm = full * lax.rsqrt(
        jnp.mean(full * full, axis=-1, keepdims=True) + 1e-6) * rms_ref[...]
    out_ref[0] = norm


def kernel(expression_values, gene_table, zero_embedding, eW1, eb1, eW2, eb2,
           eW3, eb3, mW1, mb1, mW2, mb2, cond_table, pW1, pb1, pW2, pb2,
           xW1, xb1, xW2, xb2, ctxW1, ctxb1, ctxW2, ctxb2, tf_table,
           pos_table, prefix_type, gene_type, rms_w, input_ids,
           condition_ids, padding_mask, non_tf_mask):
    gene_idx = input_ids.reshape(-1).astype(jnp.int32)
    cond_idx = condition_ids.reshape(-1).astype(jnp.int32)
    gathered, ce_rows = _sc_gather(gene_table, gene_idx, cond_table, cond_idx)
    gathered = gathered.reshape(C, G, D)
    ce3 = ce_rows.reshape(C, 1, 4 * D)

    ev3 = expression_values.reshape(C, G, 1)
    tf3 = non_tf_mask.astype(jnp.float32).reshape(C, G, 1)
    validt = (~padding_mask).astype(jnp.float32).reshape(C, 1, G)
    pos = pos_table[: G + 1]

    row = lambda b: b.reshape(1, -1)
    bf = lambda w: w.astype(jnp.bfloat16)
    W1cat = jnp.concatenate([eW1, mW1], axis=1)
    b1cat = jnp.concatenate([eb1, mb1]).reshape(1, -1)
    tf_tab_adj = tf_table + gene_type.reshape(1, D)
    weights = (W1cat, b1cat, bf(eW2), row(eb2), bf(eW3), row(eb3),
               bf(mW2), row(mb2),
               tf_tab_adj, row(zero_embedding),
               ctxW1, row(ctxb1), ctxW2, row(ctxb2),
               pW1, row(pb1), pW2, row(pb2),
               xW1, row(xb1), xW2, row(xb2),
               prefix_type.reshape(1, D), row(rms_w), pos)

    full_spec = lambda a: pl.BlockSpec(a.shape, lambda c: (0,) * a.ndim)
    in_specs = [
        pl.BlockSpec((1, G, D), lambda c: (c, 0, 0)),
        pl.BlockSpec((1, G, 1), lambda c: (c, 0, 0)),
        pl.BlockSpec((1, G, 1), lambda c: (c, 0, 0)),
        pl.BlockSpec((1, 1, G), lambda c: (c, 0, 0)),
        pl.BlockSpec((1, 1, 4 * D), lambda c: (c, 0, 0)),
    ] + [full_spec(w) for w in weights]

    out = pl.pallas_call(
        _cell_body,
        grid=(C,),
        in_specs=in_specs,
        out_specs=pl.BlockSpec((1, G + 1, D), lambda c: (c, 0, 0)),
        out_shape=jax.ShapeDtypeStruct((C, G + 1, D), jnp.float32),
    )(gathered, ev3, tf3, validt, ce3, *weights)
    return out


# speedup vs baseline: 1.2650x; 1.0148x over previous
import functools

import jax
import jax.numpy as jnp
from jax import lax
from jax.experimental import pallas as pl
from jax.experimental.pallas import tpu as pltpu
from jax.experimental.pallas import tpu_sc as plsc

C, G, D = 32, 2048, 256
GENE_V = 60000
_NC, _NS = 2, 16
_NW = _NC * _NS
_CH = 128


def _silu(x):
    return x * jax.nn.sigmoid(x)


def _sc_gather(gene_table, gene_idx, cond_table, cond_idx):
    B = gene_idx.shape[0]
    CB = cond_idx.shape[0]
    rows_w = B // _NW
    nch = rows_w // _CH

    mesh = plsc.VectorSubcoreMesh(core_axis_name="c", subcore_axis_name="s")

    @functools.partial(
        pl.kernel,
        out_type=(
            jax.ShapeDtypeStruct((B, D), jnp.float32),
            jax.ShapeDtypeStruct((CB, D), jnp.float32),
        ),
        mesh=mesh,
        scratch_types=[
            pltpu.VMEM((rows_w,), jnp.int32),
            pltpu.VMEM((_CH, D), jnp.float32),
            pltpu.VMEM((_CH, D), jnp.float32),
            pltpu.VMEM((CB,), jnp.int32),
            pltpu.VMEM((CB, D), jnp.float32),
            pltpu.SemaphoreType.DMA,
            pltpu.SemaphoreType.DMA,
            pltpu.SemaphoreType.DMA,
            pltpu.SemaphoreType.DMA,
            pltpu.SemaphoreType.DMA,
        ],
    )
    def gather_k(table_h, idx_h, ctab_h, cidx_h, out_h, cout_h,
                 idx_v, rows_a, rows_b, cidx_v, crows_v,
                 gsem_a, gsem_b, ssem_a, ssem_b, csem):
        wid = lax.axis_index("s") * _NC + lax.axis_index("c")
        base = wid * rows_w
        pltpu.sync_copy(idx_h.at[pl.ds(base, rows_w)], idx_v)

        bufs = (rows_a, rows_b)
        gsems = (gsem_a, gsem_b)
        ssems = (ssem_a, ssem_b)
        gcp = [None, None]
        scp = [None, None]
        for k in range(2):
            gcp[k] = pltpu.async_copy(
                table_h.at[idx_v.at[pl.ds(k * _CH, _CH)]], bufs[k], gsems[k])
        for k in range(nch):
            b = k % 2
            gcp[b].wait()
            scp[b] = pltpu.async_copy(
                bufs[b], out_h.at[pl.ds(base + k * _CH, _CH)], ssems[b])
            if k + 2 < nch:
                scp[b].wait()
                gcp[b] = pltpu.async_copy(
                    table_h.at[idx_v.at[pl.ds((k + 2) * _CH, _CH)]],
                    bufs[b], gsems[b])
        scp[0].wait()
        scp[1].wait()

        @pl.when(wid == 0)
        def _():
            pltpu.sync_copy(cidx_h, cidx_v)
            pltpu.async_copy(ctab_h.at[cidx_v], crows_v, csem).wait()
            pltpu.sync_copy(crows_v, cout_h)

    return gather_k(gene_table, gene_idx, cond_table, cond_idx)


def _cell_body(gene_ref, ev_ref, tf_ref, validt_ref, ce_ref,
               W1cat_ref, b1cat_ref, eW2_ref, eb2_ref, eW3_ref, eb3_ref,
               mW2_ref, mb2_ref,
               tf_tab_ref, zero_ref,
               ctxW1_ref, ctxb1_ref, ctxW2_ref, ctxb2_ref,
               pW1_ref, pb1_ref, pW2_ref, pb2_ref,
               xW1_ref, xb1_ref, xW2_ref, xb2_ref,
               prefix_type_ref, rms_ref, pos_ref, out_ref):
    f32 = jnp.float32
    bf16 = jnp.bfloat16
    NB = gene_ref.shape[0]
    R = NB * G
    v = ev_ref[...].reshape(R, 1)
    s = _silu(v * W1cat_ref[...] + b1cat_ref[...])
    h = s[:, :D]
    m = s[:, D:]
    h = _silu(jnp.dot(h.astype(bf16), eW2_ref[...],
                      preferred_element_type=f32) + eb2_ref[...])
    expr = jnp.dot(h.astype(bf16), eW3_ref[...],
                   preferred_element_type=f32) + eb3_ref[...]
    expr = jnp.where(v == 0.0, zero_ref[...], expr)
    mod = jnp.dot(m.astype(bf16), mW2_ref[...],
                  preferred_element_type=f32) + mb2_ref[...]
    scale = jax.nn.sigmoid(mod[:, :D])
    shift = mod[:, D:]
    t0 = tf_tab_ref[0:1, :]
    t1 = tf_tab_ref[1:2, :]
    tf_emb = t0 + tf_ref[...].reshape(R, 1) * (t1 - t0)
    tokens = (gene_ref[...].reshape(R, D) + expr + tf_emb) * scale + shift
    vmat = validt_ref[0]
    pooled = jnp.dot(vmat, tokens, preferred_element_type=f32) / jnp.maximum(
        jnp.sum(vmat, axis=1, keepdims=True), 1.0)
    ctx = jnp.dot(_silu(jnp.dot(pooled, ctxW1_ref[...],
                                preferred_element_type=f32) + ctxb1_ref[...]),
                  ctxW2_ref[...], preferred_element_type=f32) + ctxb2_ref[...]
    ce = ce_ref[0]
    ptok = jnp.dot(_silu(jnp.dot(ce, pW1_ref[...],
                                 preferred_element_type=f32) + pb1_ref[...]),
                   pW2_ref[...], preferred_element_type=f32) + pb2_ref[...]
    cbias = jnp.dot(_silu(jnp.dot(ce, xW1_ref[...],
                                  preferred_element_type=f32) + xb1_ref[...]),
                    xW2_ref[...], preferred_element_type=f32) + xb2_ref[...]
    prefix_rows = ptok + ctx + prefix_type_ref[...] + pos_ref[0:1, :]

    def normf(x):
        return x * lax.rsqrt(
            jnp.mean(x * x, axis=-1, keepdims=True) + 1e-6) * rms_ref[...]

    pn = normf(prefix_rows)
    for b in range(NB):
        genes_b = (tokens[b * G:(b + 1) * G] + cbias[b:b + 1]
                   + pos_ref[1:, :])
        out_ref[b, 0:1] = pn[b:b + 1]
        out_ref[b, 1:] = normf(genes_b)


def kernel(expression_values, gene_table, zero_embedding, eW1, eb1, eW2, eb2,
           eW3, eb3, mW1, mb1, mW2, mb2, cond_table, pW1, pb1, pW2, pb2,
           xW1, xb1, xW2, xb2, ctxW1, ctxb1, ctxW2, ctxb2, tf_table,
           pos_table, prefix_type, gene_type, rms_w, input_ids,
           condition_ids, padding_mask, non_tf_mask):
    gene_idx = input_ids.reshape(-1).astype(jnp.int32)
    cond_idx = condition_ids.reshape(-1).astype(jnp.int32)
    gathered, ce_rows = _sc_gather(gene_table, gene_idx, cond_table, cond_idx)
    NB = 2
    gathered = gathered.reshape(C, G, D)
    ce3 = ce_rows.reshape(C // NB, NB, 4 * D)

    ev3 = expression_values.reshape(C, G, 1)
    tf3 = non_tf_mask.astype(jnp.float32).reshape(C, G, 1)
    valid = (~padding_mask).astype(jnp.float32).reshape(C // NB, NB, G)
    eye = jnp.eye(NB, dtype=jnp.float32)[None, :, :, None]
    validt = (eye * valid[:, None, :, :]).reshape(C // NB, NB, NB * G)
    pos = pos_table[: G + 1]

    row = lambda b: b.reshape(1, -1)
    bf = lambda w: w.astype(jnp.bfloat16)
    W1cat = jnp.concatenate([eW1, mW1], axis=1)
    b1cat = jnp.concatenate([eb1, mb1]).reshape(1, -1)
    tf_tab_adj = tf_table + gene_type.reshape(1, D)
    weights = (W1cat, b1cat, bf(eW2), row(eb2), bf(eW3), row(eb3),
               bf(mW2), row(mb2),
               tf_tab_adj, row(zero_embedding),
               ctxW1, row(ctxb1), ctxW2, row(ctxb2),
               pW1, row(pb1), pW2, row(pb2),
               xW1, row(xb1), xW2, row(xb2),
               prefix_type.reshape(1, D), row(rms_w), pos)

    full_spec = lambda a: pl.BlockSpec(a.shape, lambda c: (0,) * a.ndim)
    in_specs = [
        pl.BlockSpec((NB, G, D), lambda c: (c, 0, 0)),
        pl.BlockSpec((NB, G, 1), lambda c: (c, 0, 0)),
        pl.BlockSpec((NB, G, 1), lambda c: (c, 0, 0)),
        pl.BlockSpec((1, NB, NB * G), lambda c: (c, 0, 0)),
        pl.BlockSpec((1, NB, 4 * D), lambda c: (c, 0, 0)),
    ] + [full_spec(w) for w in weights]

    out = pl.pallas_call(
        _cell_body,
        grid=(C // NB,),
        in_specs=in_specs,
        out_specs=pl.BlockSpec((NB, G + 1, D), lambda c: (c, 0, 0)),
        out_shape=jax.ShapeDtypeStruct((C, G + 1, D), jnp.float32),
    )(gathered, ev3, tf3, validt, ce3, *weights)
    return out
